# Initial kernel scaffold; baseline (speedup 1.0000x reference)
import jax
import jax.numpy as jnp
from jax.experimental import pallas as pl


def kernel(x, edge_index, W1_rel, b1, W1_root, W2_rel, b2, W2_root):
    raise NotImplementedError("write your pallas kernel here")


# speedup vs baseline: 5.2330x; 5.2330x over previous
import functools

import jax
import jax.numpy as jnp
from jax import lax
from jax.experimental import pallas as pl
from jax.experimental.pallas import tpu as pltpu
from jax.experimental.pallas import tpu_sc as plsc

N = 10000
E = 320000
D_IN = 128
D_HID = 256
D_OUT = 128

NC = 2
NS = 16
NW = NC * NS
C = 128
K = 79
E_PAD = NW * K * C
ROWS_PER_SUB = 632
N_PAD = NS * ROWS_PER_SUB


def _segsum_partial_sc(table, src_w, dst_w, zeros):
    mesh = plsc.VectorSubcoreMesh(core_axis_name="c", subcore_axis_name="s")

    @functools.partial(
        pl.kernel,
        out_type=jax.ShapeDtypeStruct((2 * N_PAD, D_IN), jnp.float32),
        mesh=mesh,
        scratch_types=[
            pltpu.VMEM((K, C), jnp.int32),
            pltpu.VMEM((K, C), jnp.int32),
            pltpu.VMEM((C, D_IN), jnp.float32),
            pltpu.VMEM_SHARED((N_PAD, D_IN), jnp.float32),
            pltpu.SemaphoreType.DMA,
        ],
    )
    def seg_kernel(table_hbm, src_hbm, dst_hbm, zeros_hbm, out_hbm,
                   src_v, dst_v, rows_v, acc_sh, sem):
        c = lax.axis_index("c")
        s = lax.axis_index("s")
        wid = s * NC + c

        pltpu.sync_copy(src_hbm.at[wid], src_v)
        pltpu.sync_copy(dst_hbm.at[wid], dst_v)

        row0 = s * ROWS_PER_SUB
        pltpu.sync_copy(zeros_hbm.at[pl.ds(row0, ROWS_PER_SUB)],
                        acc_sh.at[pl.ds(row0, ROWS_PER_SUB)])
        plsc.subcore_barrier()

        def body(i, carry):
            pltpu.async_copy(table_hbm.at[src_v.at[i]], rows_v, sem).wait()
            pltpu.sync_copy(rows_v, acc_sh.at[dst_v.at[i]], add=True)
            return carry

        lax.fori_loop(0, K, body, 0, unroll=False)

        plsc.subcore_barrier()
        pltpu.sync_copy(acc_sh.at[pl.ds(row0, ROWS_PER_SUB)],
                        out_hbm.at[pl.ds(c * N_PAD + row0, ROWS_PER_SUB)])

    return seg_kernel(table, src_w, dst_w, zeros)


def _layer1_tc(p0, p1, x, w1_rel, b1, w1_root, w2_rel):
    BLK = 2000

    def body(p0_ref, p1_ref, x_ref, w1rel_ref, b1_ref, w1root_ref, w2rel_ref,
             h_ref, g2_ref):
        agg = p0_ref[...] + p1_ref[...]
        acc = jnp.dot(agg, w1rel_ref[...], preferred_element_type=jnp.float32)
        acc += jnp.dot(x_ref[...], w1root_ref[...],
                       preferred_element_type=jnp.float32)
        h = jnp.maximum(acc + b1_ref[...], 0.0)
        h_ref[...] = h
        g2_ref[...] = jnp.dot(h, w2rel_ref[...],
                              preferred_element_type=jnp.float32)

    grid = N // BLK
    row_blk = lambda i: (i, 0)
    rep = lambda i: (0, 0)
    return pl.pallas_call(
        body,
        grid=(grid,),
        in_specs=[
            pl.BlockSpec((BLK, D_IN), row_blk),
            pl.BlockSpec((BLK, D_IN), row_blk),
            pl.BlockSpec((BLK, D_IN), row_blk),
            pl.BlockSpec((D_IN, D_HID), rep),
            pl.BlockSpec((1, D_HID), lambda i: (0, 0)),
            pl.BlockSpec((D_IN, D_HID), rep),
            pl.BlockSpec((D_HID, D_OUT), rep),
        ],
        out_specs=[
            pl.BlockSpec((BLK, D_HID), row_blk),
            pl.BlockSpec((BLK, D_OUT), row_blk),
        ],
        out_shape=[
            jax.ShapeDtypeStruct((N, D_HID), jnp.float32),
            jax.ShapeDtypeStruct((N, D_OUT), jnp.float32),
        ],
    )(p0, p1, x, w1_rel, b1.reshape(1, D_HID), w1_root, w2_rel)


def _layer2_tc(p0, p1, h, b2, w2_root):
    BLK = 2000

    def body(p0_ref, p1_ref, h_ref, b2_ref, w2root_ref, out_ref):
        acc = jnp.dot(h_ref[...], w2root_ref[...],
                      preferred_element_type=jnp.float32)
        out_ref[...] = p0_ref[...] + p1_ref[...] + b2_ref[...] + acc

    grid = N // BLK
    row_blk = lambda i: (i, 0)
    return pl.pallas_call(
        body,
        grid=(grid,),
        in_specs=[
            pl.BlockSpec((BLK, D_OUT), row_blk),
            pl.BlockSpec((BLK, D_OUT), row_blk),
            pl.BlockSpec((BLK, D_HID), row_blk),
            pl.BlockSpec((1, D_OUT), lambda i: (0, 0)),
            pl.BlockSpec((D_HID, D_OUT), lambda i: (0, 0)),
        ],
        out_specs=pl.BlockSpec((BLK, D_OUT), row_blk),
        out_shape=jax.ShapeDtypeStruct((N, D_OUT), jnp.float32),
    )(p0, p1, h, b2.reshape(1, D_OUT), w2_root)


def kernel(x, edge_index, W1_rel, b1, W1_root, W2_rel, b2, W2_root):
    ei = edge_index.astype(jnp.int32)
    pad = E_PAD - E
    src = jnp.concatenate([ei[0], jnp.zeros((pad,), jnp.int32)])
    dst = jnp.concatenate([ei[1], jnp.full((pad,), N, jnp.int32)])
    src_w = src.reshape(NW, K, C)
    dst_w = dst.reshape(NW, K, C)
    zeros = jnp.zeros((N_PAD, D_IN), jnp.float32)

    p1 = _segsum_partial_sc(x, src_w, dst_w, zeros)
    h, g2 = _layer1_tc(p1[:N], p1[N_PAD:N_PAD + N], x,
                       W1_rel, b1, W1_root, W2_rel)
    p2 = _segsum_partial_sc(g2, src_w, dst_w, zeros)
    out = _layer2_tc(p2[:N], p2[N_PAD:N_PAD + N], h, b2, W2_root)
    return out
